# Initial kernel scaffold; baseline (speedup 1.0000x reference)
import jax
import jax.numpy as jnp
from jax.experimental import pallas as pl


def kernel(batch_mat, topk_edge, embedding, W, att_i, att_j, att_em_i, att_em_j, bias, gamma, beta):
    raise NotImplementedError("write your pallas kernel here")


# speedup vs baseline: 28.8423x; 28.8423x over previous
import functools

import jax
import jax.numpy as jnp
from jax import lax
from jax.experimental import pallas as pl
from jax.experimental.pallas import tpu as pltpu
from jax.experimental.pallas import tpu_sc as plsc

N = 10000
E = 320000
C = 128
NC, NS, L = 2, 16, 16
NW = NC * NS
B = 48
NCH = 210
NGRP = NCH // 2
EPT = B * NCH
E_PAD = NW * EPT
RPT_A = 632
RPT_LAST = N - (NS - 1) * RPT_A


def _pass1_body(bm_ref, emb_ref, wt_ref, am_ref, aem_ref, x_ref, a2_ref):
    x = jnp.dot(bm_ref[...], wt_ref[...], preferred_element_type=jnp.float32)
    x_ref[...] = x
    a2_ref[...] = (
        jnp.dot(x, am_ref[...], preferred_element_type=jnp.float32)
        + jnp.dot(emb_ref[...], aem_ref[...], preferred_element_type=jnp.float32))


def _pass1(bm, emb, wt, am, aem):
    blk = 1000
    grid = N // blk
    return pl.pallas_call(
        _pass1_body,
        grid=(grid,),
        in_specs=[
            pl.BlockSpec((blk, C), lambda i: (i, 0)),
            pl.BlockSpec((blk, C), lambda i: (i, 0)),
            pl.BlockSpec((C, C), lambda i: (0, 0)),
            pl.BlockSpec((C, C), lambda i: (0, 0)),
            pl.BlockSpec((C, C), lambda i: (0, 0)),
        ],
        out_specs=[
            pl.BlockSpec((blk, C), lambda i: (i, 0)),
            pl.BlockSpec((blk, C), lambda i: (i, 0)),
        ],
        out_shape=[
            jax.ShapeDtypeStruct((N, C), jnp.float32),
            jax.ShapeDtypeStruct((N, C), jnp.float32),
        ],
    )(bm, emb, wt, am, aem)


def _sc_pass(x, eflat, ai, aj, z128):
    mesh = plsc.VectorSubcoreMesh(core_axis_name="c", subcore_axis_name="s",
                                  num_cores=NC, num_subcores=NS)

    @functools.partial(
        pl.kernel,
        out_type=(
            jax.ShapeDtypeStruct((NC, N, C), jnp.float32),
            jax.ShapeDtypeStruct((NC * N,), jnp.float32),
        ),
        mesh=mesh,
        compiler_params=pltpu.CompilerParams(needs_layout_passes=False),
        scratch_types=[
            pltpu.VMEM((N,), jnp.float32),
            pltpu.VMEM((N,), jnp.float32),
            pltpu.VMEM((2 * B,), jnp.int32),
            pltpu.VMEM((2 * B,), jnp.int32),
            pltpu.VMEM((B,), jnp.int32),
            pltpu.VMEM((B,), jnp.int32),
            pltpu.VMEM((B, C), jnp.float32),
            pltpu.VMEM((B, C), jnp.float32),
            pltpu.VMEM((B, C), jnp.float32),
            pltpu.VMEM((B, C), jnp.float32),
            pltpu.VMEM((B,), jnp.float32),
            pltpu.VMEM((B,), jnp.float32),
            pltpu.VMEM((640,), jnp.float32),
            pltpu.VMEM_SHARED((N, C), jnp.float32),
            pltpu.VMEM_SHARED((N,), jnp.float32),
            pltpu.SemaphoreType.DMA,
            pltpu.SemaphoreType.DMA,
            pltpu.SemaphoreType.DMA,
            pltpu.SemaphoreType.DMA,
        ],
    )
    def body(x_hbm, e_hbm, ai_hbm, aj_hbm, z128_hbm,
             acc_out, den_out,
             ai_v, aj_v, idx0, idx1, dst0, dst1, rows0, rows1, sbuf0, sbuf1,
             exv0, exv1, stage_v, acc_sh, den_sh, gsem0, gsem1, ssem0, ssem1):
        cid = lax.axis_index("c")
        sid = lax.axis_index("s")
        w = cid * NS + sid

        pltpu.sync_copy(ai_hbm, ai_v)
        pltpu.sync_copy(aj_hbm, aj_v)

        r0 = pl.multiple_of(sid * RPT_A, 8)
        zeros16 = jnp.zeros((L,), jnp.float32)
        for i in range(640 // L):
            stage_v[pl.ds(i * L, L)] = zeros16

        @pl.when(sid < NS - 1)
        def _zero_a():
            pltpu.sync_copy(z128_hbm.at[pl.ds(r0, RPT_A)],
                            acc_sh.at[pl.ds(r0, RPT_A)])
            pltpu.sync_copy(stage_v.at[pl.ds(0, RPT_A)],
                            den_sh.at[pl.ds(r0, RPT_A)])

        @pl.when(sid == NS - 1)
        def _zero_b():
            pltpu.sync_copy(z128_hbm.at[pl.ds(r0, RPT_LAST)],
                            acc_sh.at[pl.ds(r0, RPT_LAST)])
            pltpu.sync_copy(stage_v.at[pl.ds(0, RPT_LAST)],
                            den_sh.at[pl.ds(r0, RPT_LAST)])

        plsc.subcore_barrier()

        bufs = ((idx0, dst0, rows0, sbuf0, exv0, gsem0, ssem0),
                (idx1, dst1, rows1, sbuf1, exv1, gsem1, ssem1))

        def gather_start(idx_v, rows_v, gsem):
            pltpu.async_copy(x_hbm.at[idx_v.at[pl.ds(0, B)]], rows_v, gsem)

        def gather_wait(idx_v, rows_v, gsem):
            pltpu.make_async_copy(x_hbm.at[idx_v.at[pl.ds(0, B)]],
                                  rows_v, gsem).wait()

        def scatter_wait(dst_v, sbuf_v, exv_v, ssem):
            pltpu.make_async_copy(sbuf_v, acc_sh.at[dst_v], ssem).wait()
            pltpu.make_async_copy(exv_v, den_sh.at[dst_v], ssem).wait()

        def compute(idx_v, dst_v, rows_v, sbuf_v, exv_v):
            for k in range(B // L):
                dst_v[pl.ds(k * L, L)] = idx_v[pl.ds(B + k * L, L)]
            for k in range(B // L):
                s16 = idx_v[pl.ds(k * L, L)]
                d16 = dst_v[pl.ds(k * L, L)]
                al = (plsc.load_gather(ai_v, [d16])
                      + plsc.load_gather(aj_v, [s16]))
                al = jnp.maximum(al, 0.2 * al)
                ex = jnp.exp(al)
                ex = jnp.where(s16 == d16, jnp.float32(0.0), ex)
                exv_v[pl.ds(k * L, L)] = ex
                for b in range(L):
                    row = k * L + b
                    v = jnp.full((L,), ex[b], jnp.float32)
                    for j in range(C // L):
                        sl = (row, pl.ds(j * L, L))
                        sbuf_v[sl] = rows_v[sl] * v

        pltpu.sync_copy(e_hbm.at[w, 0], idx0)
        gather_start(idx0, rows0, gsem0)
        pltpu.sync_copy(e_hbm.at[w, 1], idx1)
        gather_start(idx1, rows1, gsem1)

        def group(cg, carry):
            for p in range(2):
                idx_v, dst_v, rows_v, sbuf_v, exv_v, gsem, ssem = bufs[p]
                ci = 2 * cg + p
                gather_wait(idx_v, rows_v, gsem)

                @pl.when(cg < NGRP - 1)
                def _prefetch_idx():
                    pltpu.async_copy(e_hbm.at[w, ci + 2], idx_v, gsem)

                @pl.when(cg > 0)
                def _drain_prev_scatter():
                    scatter_wait(dst_v, sbuf_v, exv_v, ssem)

                compute(idx_v, dst_v, rows_v, sbuf_v, exv_v)
                pltpu.async_copy(sbuf_v, acc_sh.at[dst_v], ssem, add=True)
                pltpu.async_copy(exv_v, den_sh.at[dst_v], ssem, add=True)

                @pl.when(cg < NGRP - 1)
                def _next_gather():
                    pltpu.make_async_copy(e_hbm.at[w, ci + 2],
                                          idx_v, gsem).wait()
                    gather_start(idx_v, rows_v, gsem)

            return carry

        lax.fori_loop(0, NGRP, group, 0)
        for p in range(2):
            idx_v, dst_v, rows_v, sbuf_v, exv_v, gsem, ssem = bufs[p]
            scatter_wait(dst_v, sbuf_v, exv_v, ssem)
        plsc.subcore_barrier()

        @pl.when(sid < NS - 1)
        def _out_a():
            pltpu.sync_copy(acc_sh.at[pl.ds(r0, RPT_A)],
                            acc_out.at[cid, pl.ds(r0, RPT_A)])
            doff_a = pl.multiple_of(cid * N + r0, 8)
            pltpu.sync_copy(den_sh.at[pl.ds(r0, RPT_A)],
                            stage_v.at[pl.ds(0, RPT_A)])
            pltpu.sync_copy(stage_v.at[pl.ds(0, RPT_A)],
                            den_out.at[pl.ds(doff_a, RPT_A)])

        @pl.when(sid == NS - 1)
        def _out_b():
            pltpu.sync_copy(acc_sh.at[pl.ds(r0, RPT_LAST)],
                            acc_out.at[cid, pl.ds(r0, RPT_LAST)])
            doff_b = pl.multiple_of(cid * N + r0, 8)
            pltpu.sync_copy(den_sh.at[pl.ds(r0, RPT_LAST)],
                            stage_v.at[pl.ds(0, RPT_LAST)])
            pltpu.sync_copy(stage_v.at[pl.ds(0, RPT_LAST)],
                            den_out.at[pl.ds(doff_b, RPT_LAST)])

    return body(x, eflat, ai, aj, z128)


def _pass3_body(acc_ref, den2_ref, x_ref, a2_ref, bias_ref, gamma_ref,
                beta_ref, out_ref):
    al = a2_ref[:, 0:1] + a2_ref[:, 1:2]
    exs = jnp.exp(jnp.maximum(al, 0.2 * al))
    numer = acc_ref[0] + acc_ref[1] + exs * x_ref[...]
    den = den2_ref[:, 0:1] + den2_ref[:, 1:2] + exs
    o = numer / jnp.maximum(den, jnp.float32(1e-30)) + bias_ref[...]
    mean = jnp.mean(o, axis=0, keepdims=True)
    var = jnp.mean((o - mean) * (o - mean), axis=0, keepdims=True)
    o = (o - mean) / jnp.sqrt(var + 1e-5) * gamma_ref[...] + beta_ref[...]
    out_ref[...] = jnp.maximum(o, jnp.float32(0.0))


def _pass3(acc, den2, x, a2, bias, gamma, beta):
    return pl.pallas_call(
        _pass3_body,
        out_shape=jax.ShapeDtypeStruct((N, C), jnp.float32),
    )(acc, den2, x, a2, bias, gamma, beta)


def kernel(batch_mat, topk_edge, embedding, W, att_i, att_j, att_em_i,
           att_em_j, bias, gamma, beta):
    wt = W.T
    am = jnp.zeros((C, C), jnp.float32).at[:, 0].set(att_i).at[:, 1].set(att_j)
    aem = (jnp.zeros((C, C), jnp.float32)
           .at[:, 0].set(att_em_i).at[:, 1].set(att_em_j))
    x, a2 = _pass1(batch_mat, embedding, wt, am, aem)

    pad = jnp.zeros((2, E_PAD - E), jnp.int32)
    edges = jnp.concatenate([topk_edge, pad], axis=1)
    srcr = edges[0].reshape(NW, NCH, 1, B)
    dstr = edges[1].reshape(NW, NCH, 1, B)
    eflat = jnp.concatenate([srcr, dstr], axis=2).reshape(NW, NCH, 2 * B)
    ai = a2[:, 0]
    aj = a2[:, 1]
    z128 = jnp.zeros((N, C), jnp.float32)
    acc, den = _sc_pass(x, eflat, ai, aj, z128)

    return _pass3(acc, den.reshape(NC, N).T, x, a2, bias.reshape(1, C),
                  gamma.reshape(1, C), beta.reshape(1, C))
